# baseline (speedup 1.0000x reference)
import functools

import jax
import jax.numpy as jnp
import numpy as np
from jax import lax
from jax.experimental import pallas as pl
from jax.experimental.pallas import tpu as pltpu
from jax.experimental.pallas import tpu_sc as plsc

VOCAB = 100000
DIM = 128
BATCH = 4096
SEQ = 50
HID = 128
ALTS = 64

NC = 2
NS = 16
NW = NC * NS
BPW = BATCH // NW
IPW = BPW * SEQ
CHUNK = 64
NCHUNK = IPW // CHUNK
NSLOT = 10
LEAD = 5
SPB = BPW // CHUNK


def _sc_pool_sum(x_resh, dest_all, table):
    mesh = plsc.VectorSubcoreMesh(core_axis_name="c", subcore_axis_name="s")

    @functools.partial(
        pl.kernel,
        out_type=jax.ShapeDtypeStruct((BATCH, DIM), jnp.float32),
        mesh=mesh,
        scratch_types=[
            pltpu.VMEM((NCHUNK, CHUNK), jnp.int32),
            pltpu.VMEM((SPB, CHUNK), jnp.int32),
            *[pltpu.VMEM((CHUNK, DIM), jnp.float32) for _ in range(NSLOT)],
            pltpu.VMEM_SHARED((NS * BPW, DIM), jnp.float32),
            *[pltpu.SemaphoreType.DMA for _ in range(2 * NSLOT)],
        ],
    )
    def k(x_hbm, dest_hbm, table_hbm, out_hbm, idx_v, dest_v, *rest):
        rows = rest[:NSLOT]
        acc_sh = rest[NSLOT]
        gsems = rest[NSLOT + 1:NSLOT + 1 + NSLOT]
        ssems = rest[NSLOT + 1 + NSLOT:]
        c = lax.axis_index("c")
        s = lax.axis_index("s")
        w = c * NS + s
        pltpu.sync_copy(x_hbm.at[w], idx_v)
        pltpu.sync_copy(dest_hbm.at[s], dest_v)

        for b in range(LEAD):
            pltpu.async_copy(table_hbm.at[idx_v.at[b]], rows[b], gsems[b])

        def body(i, carry):
            for b in range(NSLOT):
                g = NSLOT * i + b
                pltpu.make_async_copy(table_hbm.at[idx_v.at[g]], rows[b], gsems[b]).wait()

                @pl.when(g < SPB)
                def _():
                    pltpu.async_copy(rows[b], acc_sh.at[dest_v.at[b % SPB]],
                                     ssems[b], add=False)

                @pl.when(g >= SPB)
                def _():
                    pltpu.async_copy(rows[b], acc_sh.at[dest_v.at[b % SPB]],
                                     ssems[b], add=True)

                nb = (b + LEAD) % NSLOT

                @pl.when((g + LEAD < NCHUNK) & (g >= LEAD))
                def _():
                    pltpu.make_async_copy(
                        rows[nb], acc_sh.at[dest_v.at[nb % SPB]], ssems[nb]).wait()

                @pl.when(g + LEAD < NCHUNK)
                def _():
                    pltpu.async_copy(table_hbm.at[idx_v.at[g + LEAD]], rows[nb], gsems[nb])
            return carry

        lax.fori_loop(0, NCHUNK // NSLOT, body, 0)
        for b in range(NSLOT):
            pltpu.make_async_copy(rows[b], acc_sh.at[dest_v.at[b % SPB]],
                                  ssems[b]).wait()
        pltpu.sync_copy(acc_sh.at[pl.ds(s * BPW, BPW)],
                        out_hbm.at[pl.ds(w * BPW, BPW)])

    return k(x_resh, dest_all, table)


def _tc_mlp(pooled, W1, b1, W2, b2, W3, b3):
    def body(p_ref, w1_ref, b1_ref, w2_ref, b2_ref, w3_ref, b3_ref, o_ref):
        h = p_ref[...] * np.float32(1.0 / SEQ)
        h = jnp.maximum(
            jnp.dot(h, w1_ref[...], preferred_element_type=jnp.float32) + b1_ref[...], 0.0)
        h = jnp.maximum(
            jnp.dot(h, w2_ref[...], preferred_element_type=jnp.float32) + b2_ref[...], 0.0)
        h = jnp.maximum(
            jnp.dot(h, w2_ref[...], preferred_element_type=jnp.float32) + b2_ref[...], 0.0)
        o_ref[...] = (
            jnp.dot(h, w3_ref[...], preferred_element_type=jnp.float32) + b3_ref[...])

    mb = 512
    return pl.pallas_call(
        body,
        grid=(BATCH // mb,),
        in_specs=[
            pl.BlockSpec((mb, DIM), lambda i: (i, 0)),
            pl.BlockSpec((DIM, HID), lambda i: (0, 0)),
            pl.BlockSpec((1, HID), lambda i: (0, 0)),
            pl.BlockSpec((HID, HID), lambda i: (0, 0)),
            pl.BlockSpec((1, HID), lambda i: (0, 0)),
            pl.BlockSpec((HID, ALTS), lambda i: (0, 0)),
            pl.BlockSpec((1, ALTS), lambda i: (0, 0)),
        ],
        out_specs=pl.BlockSpec((mb, ALTS), lambda i: (i, 0)),
        out_shape=jax.ShapeDtypeStruct((BATCH, ALTS), jnp.float32),
    )(pooled, W1, b1.reshape(1, HID), W2, b2.reshape(1, HID), W3, b3.reshape(1, ALTS))


def kernel(x, table, W1, b1, W2, b2, W3, b3):
    x_resh = (x.astype(jnp.int32).reshape(NW, BPW, SEQ)
              .transpose(0, 2, 1).reshape(NW, NCHUNK, CHUNK))
    dest_all = (np.arange(NS, dtype=np.int32)[:, None] * BPW
                + np.arange(BPW, dtype=np.int32)[None, :]).reshape(NS, SPB, CHUNK)
    dest_all = jnp.asarray(dest_all)
    pooled = _sc_pool_sum(x_resh, dest_all, table)
    return _tc_mlp(pooled, W1, b1, W2, b2, W3, b3)


# speedup vs baseline: 1.0940x; 1.0940x over previous
import functools

import jax
import jax.numpy as jnp
import numpy as np
from jax import lax
from jax.experimental import pallas as pl
from jax.experimental.pallas import tpu as pltpu
from jax.experimental.pallas import tpu_sc as plsc

VOCAB = 100000
DIM = 128
BATCH = 4096
SEQ = 50
HID = 128
ALTS = 64

NC = 2
NS = 16
NW = NC * NS
BPW = BATCH // NW
IPW = BPW * SEQ
CHUNK = 128
NCHUNK = IPW // CHUNK
NBUF = 5


def _sc_pool_sum(x_resh, dest_all, table):
    mesh = plsc.VectorSubcoreMesh(core_axis_name="c", subcore_axis_name="s")

    @functools.partial(
        pl.kernel,
        out_type=jax.ShapeDtypeStruct((BATCH, DIM), jnp.float32),
        mesh=mesh,
        scratch_types=[
            pltpu.VMEM((NCHUNK, CHUNK), jnp.int32),
            pltpu.VMEM((1, CHUNK), jnp.int32),
            *[pltpu.VMEM((CHUNK, DIM), jnp.float32) for _ in range(NBUF)],
            pltpu.VMEM_SHARED((NS * BPW, DIM), jnp.float32),
            *[pltpu.SemaphoreType.DMA for _ in range(NBUF)],
        ],
    )
    def k(x_hbm, dest_hbm, table_hbm, out_hbm, idx_v, dest_v, *rest):
        rows = rest[:NBUF]
        acc_sh = rest[NBUF]
        sems = rest[NBUF + 1:]
        c = lax.axis_index("c")
        s = lax.axis_index("s")
        w = c * NS + s
        pltpu.sync_copy(x_hbm.at[w], idx_v)
        pltpu.sync_copy(dest_hbm.at[s], dest_v)

        for b in range(NBUF):
            pltpu.async_copy(table_hbm.at[idx_v.at[b]], rows[b], sems[b])

        def body(i, carry):
            for b in range(NBUF):
                g = NBUF * i + b
                pltpu.make_async_copy(table_hbm.at[idx_v.at[g]], rows[b], sems[b]).wait()

                @pl.when(g == 0)
                def _():
                    pltpu.sync_copy(rows[b], acc_sh.at[dest_v.at[0]], add=False)

                @pl.when(g > 0)
                def _():
                    pltpu.sync_copy(rows[b], acc_sh.at[dest_v.at[0]], add=True)

                @pl.when(g + NBUF < NCHUNK)
                def _():
                    pltpu.async_copy(table_hbm.at[idx_v.at[g + NBUF]], rows[b], sems[b])
            return carry

        lax.fori_loop(0, NCHUNK // NBUF, body, 0)
        pltpu.sync_copy(acc_sh.at[pl.ds(s * BPW, BPW)],
                        out_hbm.at[pl.ds(w * BPW, BPW)])

    return k(x_resh, dest_all, table)


def _tc_mlp(pooled, W1, b1, W2, b2, W3, b3):
    def body(p_ref, w1_ref, b1_ref, w2_ref, b2_ref, w3_ref, b3_ref, o_ref):
        h = p_ref[...] * np.float32(1.0 / SEQ)
        h = jnp.maximum(
            jnp.dot(h, w1_ref[...], preferred_element_type=jnp.float32) + b1_ref[...], 0.0)
        h = jnp.maximum(
            jnp.dot(h, w2_ref[...], preferred_element_type=jnp.float32) + b2_ref[...], 0.0)
        h = jnp.maximum(
            jnp.dot(h, w2_ref[...], preferred_element_type=jnp.float32) + b2_ref[...], 0.0)
        o_ref[...] = (
            jnp.dot(h, w3_ref[...], preferred_element_type=jnp.float32) + b3_ref[...])

    mb = 1024
    return pl.pallas_call(
        body,
        grid=(BATCH // mb,),
        in_specs=[
            pl.BlockSpec((mb, DIM), lambda i: (i, 0)),
            pl.BlockSpec((DIM, HID), lambda i: (0, 0)),
            pl.BlockSpec((1, HID), lambda i: (0, 0)),
            pl.BlockSpec((HID, HID), lambda i: (0, 0)),
            pl.BlockSpec((1, HID), lambda i: (0, 0)),
            pl.BlockSpec((HID, ALTS), lambda i: (0, 0)),
            pl.BlockSpec((1, ALTS), lambda i: (0, 0)),
        ],
        out_specs=pl.BlockSpec((mb, ALTS), lambda i: (i, 0)),
        out_shape=jax.ShapeDtypeStruct((BATCH, ALTS), jnp.float32),
    )(pooled, W1, b1.reshape(1, HID), W2, b2.reshape(1, HID), W3, b3.reshape(1, ALTS))


def kernel(x, table, W1, b1, W2, b2, W3, b3):
    x_resh = x.astype(jnp.int32).reshape(NW, BPW, SEQ).transpose(0, 2, 1)
    dest_all = (np.arange(NS, dtype=np.int32)[:, None] * BPW
                + np.arange(BPW, dtype=np.int32)[None, :]).reshape(NS, 1, CHUNK)
    dest_all = jnp.asarray(dest_all)
    pooled = _sc_pool_sum(x_resh, dest_all, table)
    return _tc_mlp(pooled, W1, b1, W2, b2, W3, b3)


# speedup vs baseline: 1.1127x; 1.0171x over previous
import functools

import jax
import jax.numpy as jnp
import numpy as np
from jax import lax
from jax.experimental import pallas as pl
from jax.experimental.pallas import tpu as pltpu
from jax.experimental.pallas import tpu_sc as plsc

VOCAB = 100000
DIM = 128
BATCH = 4096
SEQ = 50
HID = 128
ALTS = 64

NC = 2
NS = 16
NW = NC * NS
BPW = BATCH // NW
IPW = BPW * SEQ
CHUNK = 128
NCHUNK = IPW // CHUNK
NBUF = 5


def _sc_pool_sum(x_resh, dest_all, table):
    mesh = plsc.VectorSubcoreMesh(core_axis_name="c", subcore_axis_name="s")

    @functools.partial(
        pl.kernel,
        out_type=jax.ShapeDtypeStruct((BATCH, DIM), jnp.float32),
        mesh=mesh,
        scratch_types=[
            pltpu.VMEM((NCHUNK, CHUNK), jnp.int32),
            pltpu.VMEM((1, CHUNK), jnp.int32),
            *[pltpu.VMEM((CHUNK, DIM), jnp.float32) for _ in range(NBUF)],
            pltpu.VMEM_SHARED((NS * BPW, DIM), jnp.float32),
            *[pltpu.SemaphoreType.DMA for _ in range(NBUF)],
        ],
    )
    def k(x_hbm, dest_hbm, table_hbm, out_hbm, idx_v, dest_v, *rest):
        rows = rest[:NBUF]
        acc_sh = rest[NBUF]
        sems = rest[NBUF + 1:]
        c = lax.axis_index("c")
        s = lax.axis_index("s")
        w = c * NS + s
        pltpu.sync_copy(x_hbm.at[w], idx_v)
        pltpu.sync_copy(dest_hbm.at[s], dest_v)

        for b in range(NBUF):
            pltpu.async_copy(table_hbm.at[idx_v.at[b]], rows[b], sems[b])

        def body(i, carry):
            for b in range(NBUF):
                g = NBUF * i + b
                pltpu.make_async_copy(table_hbm.at[idx_v.at[g]], rows[b], sems[b]).wait()

                @pl.when(g == 0)
                def _():
                    pltpu.sync_copy(rows[b], acc_sh.at[dest_v.at[0]], add=False)

                @pl.when(g > 0)
                def _():
                    pltpu.sync_copy(rows[b], acc_sh.at[dest_v.at[0]], add=True)

                @pl.when(g + NBUF < NCHUNK)
                def _():
                    pltpu.async_copy(table_hbm.at[idx_v.at[g + NBUF]], rows[b], sems[b])
            return carry

        lax.fori_loop(0, NCHUNK // NBUF, body, 0)
        pltpu.sync_copy(acc_sh.at[pl.ds(s * BPW, BPW)],
                        out_hbm.at[pl.ds(w * BPW, BPW)])

    return k(x_resh, dest_all, table)


def _tc_mlp(pooled, W1, b1, W2, b2, W3, b3):
    def body(p_ref, w1_ref, b1_ref, w2_ref, b2_ref, w3_ref, b3_ref, o_ref):
        h = p_ref[...] * np.float32(1.0 / SEQ)
        h = jnp.maximum(
            jnp.dot(h, w1_ref[...], preferred_element_type=jnp.float32) + b1_ref[...], 0.0)
        h = jnp.maximum(
            jnp.dot(h, w2_ref[...], preferred_element_type=jnp.float32) + b2_ref[...], 0.0)
        h = jnp.maximum(
            jnp.dot(h, w2_ref[...], preferred_element_type=jnp.float32) + b2_ref[...], 0.0)
        o_ref[...] = (
            jnp.dot(h, w3_ref[...], preferred_element_type=jnp.float32) + b3_ref[...])

    mb = 2048
    return pl.pallas_call(
        body,
        grid=(BATCH // mb,),
        in_specs=[
            pl.BlockSpec((mb, DIM), lambda i: (i, 0)),
            pl.BlockSpec((DIM, HID), lambda i: (0, 0)),
            pl.BlockSpec((1, HID), lambda i: (0, 0)),
            pl.BlockSpec((HID, HID), lambda i: (0, 0)),
            pl.BlockSpec((1, HID), lambda i: (0, 0)),
            pl.BlockSpec((HID, ALTS), lambda i: (0, 0)),
            pl.BlockSpec((1, ALTS), lambda i: (0, 0)),
        ],
        out_specs=pl.BlockSpec((mb, ALTS), lambda i: (i, 0)),
        out_shape=jax.ShapeDtypeStruct((BATCH, ALTS), jnp.float32),
    )(pooled, W1, b1.reshape(1, HID), W2, b2.reshape(1, HID), W3, b3.reshape(1, ALTS))


def kernel(x, table, W1, b1, W2, b2, W3, b3):
    x_resh = x.astype(jnp.int32).reshape(NW, BPW, SEQ).transpose(0, 2, 1)
    dest_all = (np.arange(NS, dtype=np.int32)[:, None] * BPW
                + np.arange(BPW, dtype=np.int32)[None, :]).reshape(NS, 1, CHUNK)
    dest_all = jnp.asarray(dest_all)
    pooled = _sc_pool_sum(x_resh, dest_all, table)
    return _tc_mlp(pooled, W1, b1, W2, b2, W3, b3)
